# baseline (speedup 1.0000x reference)
import jax
import jax.numpy as jnp
from jax import lax
from jax.experimental import pallas as pl
from jax.experimental.pallas import tpu as pltpu

_ROWS = 32 * 512
_COLS = 512
_BLOCK_ROWS = 1024
_NSLICE = 4
_GRID = _ROWS // _BLOCK_ROWS // _NSLICE
_STRIP = 32
_TOTAL = float(_ROWS * _COLS)


def _half_sums(r, t):
    d = r - t
    d2 = d * d
    mf = t > 0.0
    md2 = jnp.where(mf, d2, 0.0)
    return (
        jnp.sum(md2),
        jnp.sum(d2),
        jnp.sum(jnp.where(mf, 1.0, 0.0)),
    )


def _body(r1_ref, r2_ref, r3_ref, r4_ref, t1_ref, t2_ref, t3_ref, t4_ref,
          o0, o1, o2, o3, o4, o5, o6, o7, acc_ref):
    i = pl.program_id(0)

    f1, s1, c1 = _half_sums(r1_ref[...], t1_ref[...])
    f2, s2, c2 = _half_sums(r2_ref[...], t2_ref[...])
    f3, s3, c3 = _half_sums(r3_ref[...], t3_ref[...])
    f4, s4, c4 = _half_sums(r4_ref[...], t4_ref[...])
    fsum = (f1 + f2) + (f3 + f4)
    tsum = (s1 + s2) + (s3 + s4)
    fcnt = (c1 + c2) + (c3 + c4)

    @pl.when(i == 0)
    def _():
        acc_ref[0] = fsum
        acc_ref[1] = tsum
        acc_ref[2] = fcnt

    @pl.when(i > 0)
    def _():
        acc_ref[0] += fsum
        acc_ref[1] += tsum
        acc_ref[2] += fcnt

    @pl.when(i == _GRID - 1)
    def _():
        sf = acc_ref[0]
        st = acc_ref[1]
        nf = acc_ref[2]
        su = st - sf
        nu = _TOTAL - nf
        flood = jnp.where(nf > 0.0, sf / jnp.maximum(nf, 1.0), 0.0)
        unflood = jnp.where(nu > 0.0, su / jnp.maximum(nu, 1.0), 0.0)
        loss_reg = 20.0 * flood + unflood
        o0[0] = 2.0 * loss_reg
        o1[0] = 2.0 * loss_reg
        o2[0] = 2.0 * flood
        o3[0] = 2.0 * unflood
        o4[0] = loss_reg
        o5[0] = flood
        o6[0] = unflood
        o7[0] = 0.0


@jax.jit
def _run(reg, targets):
    reg2 = reg.reshape(_ROWS, _COLS)
    tgt2 = targets.reshape(_ROWS, _COLS)
    sds = jax.ShapeDtypeStruct((1,), jnp.float32)
    outs = pl.pallas_call(
        _body,
        grid=(_GRID,),
        in_specs=[
            pl.BlockSpec((_BLOCK_ROWS, _COLS), lambda i: (i, 0)),
            pl.BlockSpec((_BLOCK_ROWS, _COLS), lambda i: (i + _GRID, 0)),
            pl.BlockSpec((_BLOCK_ROWS, _COLS), lambda i: (i + 2 * _GRID, 0)),
            pl.BlockSpec((_BLOCK_ROWS, _COLS), lambda i: (i + 3 * _GRID, 0)),
            pl.BlockSpec((_BLOCK_ROWS, _COLS), lambda i: (i, 0)),
            pl.BlockSpec((_BLOCK_ROWS, _COLS), lambda i: (i + _GRID, 0)),
            pl.BlockSpec((_BLOCK_ROWS, _COLS), lambda i: (i + 2 * _GRID, 0)),
            pl.BlockSpec((_BLOCK_ROWS, _COLS), lambda i: (i + 3 * _GRID, 0)),
        ],
        out_specs=[pl.BlockSpec(memory_space=pltpu.SMEM)] * 8,
        out_shape=[sds] * 8,
        scratch_shapes=[pltpu.SMEM((4,), jnp.float32)],
        compiler_params=pltpu.CompilerParams(
            dimension_semantics=("arbitrary",)
        ),
    )(reg2, reg2, reg2, reg2, tgt2, tgt2, tgt2, tgt2)
    return (
        outs[0],
        outs[1].reshape(()),
        outs[2].reshape(()),
        outs[3].reshape(()),
        outs[4].reshape(()),
        outs[5].reshape(()),
        outs[6].reshape(()),
        outs[7],
    )


def kernel(reg, targets):
    return _run(reg, targets)
